# baseline (speedup 1.0000x reference)
import functools

import jax
import jax.numpy as jnp
from jax import lax
from jax.experimental import pallas as pl
from jax.experimental.pallas import tpu as pltpu
from jax.experimental.pallas import tpu_sc as plsc

N = 10000
E = 320000
D = 128
R = 500
B = 1024

NC = 2
NS = 16
NW = NC * NS
EPT = E // NW
C = 80
NCH = EPT // C
ROWS_PT = 624
ROWS_LAST = N - ROWS_PT * (NS - 1)

NEG_SENTINEL = -3.0e38


def _edge_kernel_body(ent_hbm, rel_hbm, idx_hbm, z_hbm, out_hbm,
                      idx0, idx1, idx2, idx3, rows0, rows1, relr0, relr1,
                      agg_sp, rel_sp,
                      sem0, sem1, semr0, semr1, semi0, semi1, semi2, semi3):
  idxs = (idx0, idx1, idx2, idx3)
  rows = (rows0, rows1)
  relr = (relr0, relr1)
  sems = (sem0, sem1)
  semsr = (semr0, semr1)
  semsi = (semi0, semi1, semi2, semi3)

  cid = lax.axis_index("c")
  sid = lax.axis_index("s")
  w = sid * NC + cid

  @pl.when(sid == 0)
  def _():
    pltpu.sync_copy(rel_hbm, rel_sp)

  for q in range(4):
    pltpu.async_copy(idx_hbm.at[w, q], idxs[q], semsi[q])

  def _per_tile_rows(fn):
    @pl.when(sid < NS - 1)
    def _():
      fn(sid * ROWS_PT, ROWS_PT)

    @pl.when(sid == NS - 1)
    def _():
      fn(ROWS_PT * (NS - 1), ROWS_LAST)

  def _zero(base, size):
    pltpu.sync_copy(z_hbm.at[pl.ds(base, size)],
                    agg_sp.at[pl.ds(base, size)])
  _per_tile_rows(_zero)

  for b in range(2):
    pltpu.make_async_copy(idx_hbm.at[w, b], idxs[b], semsi[b]).wait()
    pltpu.async_copy(ent_hbm.at[idxs[b].at[0]], rows[b], sems[b])

  plsc.subcore_barrier()

  for b in range(2):
    pltpu.async_copy(rel_sp.at[idxs[b].at[2]], relr[b], semsr[b])

  def compute_chunk(rows_ref, relr_ref):
    def edge_body(e, _):
      for k in range(8):
        sl = pl.ds(k * 16, 16)
        rows_ref[e, sl] = rows_ref[e, sl] * relr_ref[e, sl]
      return 0
    lax.fori_loop(0, C, edge_body, 0, unroll=8)

  def do_chunk(jb, b, qn, qi):
    pltpu.make_async_copy(ent_hbm.at[idxs[qi].at[0]], rows[b],
                          sems[b]).wait()
    pltpu.make_async_copy(rel_sp.at[idxs[qi].at[2]], relr[b],
                          semsr[b]).wait()
    compute_chunk(rows[b], relr[b])
    pltpu.sync_copy(rows[b], agg_sp.at[idxs[qi].at[1]], add=True)

    @pl.when(jb + 2 < NCH)
    def _():
      pltpu.make_async_copy(idx_hbm.at[w, jb + 2], idxs[qn], semsi[qn]).wait()
      pltpu.async_copy(ent_hbm.at[idxs[qn].at[0]], rows[b], sems[b])
      pltpu.async_copy(rel_sp.at[idxs[qn].at[2]], relr[b], semsr[b])

    @pl.when(jb + 4 < NCH)
    def _():
      pltpu.async_copy(idx_hbm.at[w, jb + 4], idxs[qi], semsi[qi])

  def quad_body(p, _):
    jb = 4 * p
    do_chunk(jb, 0, 2, 0)
    do_chunk(jb + 1, 1, 3, 1)
    do_chunk(jb + 2, 0, 0, 2)
    do_chunk(jb + 3, 1, 1, 3)
    return 0

  lax.fori_loop(0, NCH // 4, quad_body, 0)
  for t in range(NCH - (NCH // 4) * 4):
    jb = (NCH // 4) * 4 + t
    do_chunk(jb, jb % 2, (jb + 2) % 4, jb % 4)

  plsc.subcore_barrier()

  def _writeout(base, size):
    pltpu.sync_copy(agg_sp.at[pl.ds(base, size)],
                    out_hbm.at[cid, pl.ds(base, size)])
  _per_tile_rows(_writeout)


def _make_edge_kernel():
  mesh = plsc.VectorSubcoreMesh(core_axis_name="c", subcore_axis_name="s")
  scratch = (
      [pltpu.VMEM((3, C), jnp.int32) for _ in range(4)]
      + [pltpu.VMEM((C, D), jnp.float32) for _ in range(4)]
      + [pltpu.VMEM_SHARED((N, D), jnp.float32),
         pltpu.VMEM_SHARED((R, D), jnp.float32)]
      + [pltpu.SemaphoreType.DMA] * 8)
  return pl.kernel(
      _edge_kernel_body,
      out_type=[jax.ShapeDtypeStruct((NC, N, D), jnp.float32)],
      mesh=mesh, scratch_types=scratch, name="compgcn_edges")


@functools.lru_cache(maxsize=None)
def _get_edge_kernel():
  return _make_edge_kernel()


def _dense_body(layer, do_relu, p_ref, deg_ref, ent_ref, rel_ref, q_ref,
                w_ref, wl_ref, wr_ref, lr_ref, b_ref, g_ref, be_ref,
                oent_ref, orel_ref):
  pre = p_ref[0] + p_ref[1]
  deg = deg_ref[0, :, 0:1] + deg_ref[1, :, 0:1]
  agg = jnp.dot(pre, w_ref[...], preferred_element_type=jnp.float32)
  agg = agg / jnp.maximum(deg, 1.0)
  loop_msg = jnp.dot(ent_ref[...] * lr_ref[...], wl_ref[...],
                     preferred_element_type=jnp.float32)
  out = 0.5 * (agg + loop_msg) + b_ref[...]
  m = jnp.mean(out, axis=0, keepdims=True)
  v = jnp.mean((out - m) * (out - m), axis=0, keepdims=True)
  out = g_ref[...] * (out - m) * lax.rsqrt(v + 1e-5) + be_ref[...]
  out = jnp.where(q_ref[...] <= layer, ent_ref[...], out)
  if do_relu:
    out = jnp.maximum(out, 0.0)
  oent_ref[...] = out
  orel_ref[...] = jnp.dot(rel_ref[...], wr_ref[...],
                          preferred_element_type=jnp.float32)


def _make_dense(layer, do_relu):
  return pl.pallas_call(
      functools.partial(_dense_body, layer, do_relu),
      out_shape=[jax.ShapeDtypeStruct((N, D), jnp.float32),
                 jax.ShapeDtypeStruct((R, D), jnp.float32)],
      name=f"compgcn_dense_{layer}")


_dense_layers = [_make_dense(0, True), _make_dense(1, True),
                 _make_dense(2, False)]

SEG_ROWS = 320
SEG_GROUPS = SEG_ROWS // 16


def _segmax_body(ent_hbm, bid_hbm, sent_hbm, out_hbm,
                 bid_v, rows_v, stag_v, sent_v):
  cid = lax.axis_index("c")
  sid = lax.axis_index("s")
  w = sid * NC + cid
  base = w * SEG_ROWS
  is_last = w == NW - 1
  mysize = jnp.where(is_last, N - (NW - 1) * SEG_ROWS, SEG_ROWS)
  ngroups = mysize // 16

  pltpu.sync_copy(sent_hbm, sent_v)
  for blk in range(B // 64):
    pltpu.sync_copy(sent_v, out_hbm.at[w, pl.ds(blk * 64, 64)])

  @pl.when(jnp.logical_not(is_last))
  def _():
    pltpu.sync_copy(bid_hbm.at[pl.ds(base, SEG_ROWS)], bid_v)
    pltpu.sync_copy(ent_hbm.at[pl.ds(base, SEG_ROWS)], rows_v)

  @pl.when(is_last)
  def _():
    tail = N - (NW - 1) * SEG_ROWS
    pltpu.sync_copy(bid_hbm.at[pl.ds(base, tail)], bid_v.at[pl.ds(0, tail)])
    pltpu.sync_copy(ent_hbm.at[pl.ds(base, tail)], rows_v.at[pl.ds(0, tail)])

  for k in range(8):
    stag_v[pl.ds(k * 16, 16)] = jnp.full((16,), NEG_SENTINEL, jnp.float32)

  prev0 = bid_v[pl.ds(0, 16)][0]

  def group_scan(g, prev):
    ids16 = bid_v[pl.ds(g * 16, 16)]

    def do_lane(lane, prev_lane):
      rid = ids16[lane]
      changed = jnp.logical_and(rid != prev_lane, g * 16 + lane < mysize)

      @pl.when(changed)
      def _():
        pltpu.sync_copy(stag_v, out_hbm.at[w, prev_lane])

      @pl.when(g * 16 + lane < mysize)
      def _():
        r = g * 16 + lane
        for k in range(8):
          sl = pl.ds(k * 16, 16)
          row = rows_v[r, sl]
          cur = stag_v[sl]
          stag_v[sl] = jnp.where(changed, row, jnp.maximum(cur, row))
      return jnp.where(g * 16 + lane < mysize, rid, prev_lane)

    for lane in range(16):
      prev = do_lane(lane, prev)
    return prev

  prev_last = lax.fori_loop(0, SEG_GROUPS, group_scan, prev0)
  pltpu.sync_copy(stag_v, out_hbm.at[w, prev_last])


@functools.lru_cache(maxsize=None)
def _get_segmax_kernel():
  return pl.kernel(
      _segmax_body,
      out_type=[jax.ShapeDtypeStruct((NW, B, D), jnp.float32)],
      mesh=plsc.VectorSubcoreMesh(core_axis_name="c", subcore_axis_name="s"),
      scratch_types=[
          pltpu.VMEM((SEG_ROWS,), jnp.int32),
          pltpu.VMEM((SEG_ROWS, D), jnp.float32),
          pltpu.VMEM((D,), jnp.float32),
          pltpu.VMEM((64, D), jnp.float32),
      ],
      name="segment_max_partials")


def _segreduce_body(p_ref, out_ref):
  m = jnp.max(p_ref[...], axis=0)
  out_ref[...] = jnp.where(m <= NEG_SENTINEL, 0.0, m)


@functools.lru_cache(maxsize=None)
def _get_segreduce():
  return pl.pallas_call(
      _segreduce_body,
      out_shape=jax.ShapeDtypeStruct((B, D), jnp.float32),
      name="segment_max_reduce")


def kernel(ent_embed, rel_embed, edge_index, edge_type, q_diameters,
           batch_idx, target_idx, W, W_loop, W_rel, loop_rel, bias,
           bn_gamma, bn_beta):
  src = edge_index[0].astype(jnp.int32).reshape(NW, NCH, C)
  dst = edge_index[1].astype(jnp.int32).reshape(NW, NCH, C)
  typ = edge_type.astype(jnp.int32).reshape(NW, NCH, C)
  idx_pack = jnp.stack([src, dst, typ], axis=2)
  z = jnp.zeros((N, D), jnp.float32)
  q2 = q_diameters.astype(jnp.int32).reshape(N, 1)
  lr = loop_rel.reshape(1, D)
  b2 = bias.reshape(1, D)
  g2 = bn_gamma.reshape(1, D)
  be2 = bn_beta.reshape(1, D)

  ent = ent_embed
  rel = rel_embed
  (deg16,) = _get_edge_kernel()(jnp.ones((N, D), jnp.float32),
                                jnp.ones((R, D), jnp.float32), idx_pack, z)
  for layer in range(3):
    (parts,) = _get_edge_kernel()(ent, rel, idx_pack, z)
    ent, rel = _dense_layers[layer](parts, deg16, ent, rel, q2,
                                    W, W_loop, W_rel, lr, b2, g2, be2)

  sent = jnp.full((64, D), NEG_SENTINEL, jnp.float32)
  (partials,) = _get_segmax_kernel()(ent, batch_idx.astype(jnp.int32), sent)
  return _get_segreduce()(partials)


# speedup vs baseline: 1.0196x; 1.0196x over previous
import functools

import jax
import jax.numpy as jnp
from jax import lax
from jax.experimental import pallas as pl
from jax.experimental.pallas import tpu as pltpu
from jax.experimental.pallas import tpu_sc as plsc

N = 10000
E = 320000
D = 128
R = 500
B = 1024

NC = 2
NS = 16
NW = NC * NS
EPT = E // NW
C = 80
NCH = EPT // C
ROWS_PT = 624
ROWS_LAST = N - ROWS_PT * (NS - 1)

NEG_SENTINEL = -3.0e38


def _edge_kernel_body(ent_hbm, rel_hbm, idx_hbm, z_hbm, out_hbm,
                      idx0, idx1, idx2, idx3, rows0, rows1, relr0, relr1,
                      agg_sp, rel_sp,
                      sem0, sem1, semr0, semr1, semi0, semi1, semi2, semi3):
  idxs = (idx0, idx1, idx2, idx3)
  rows = (rows0, rows1)
  relr = (relr0, relr1)
  sems = (sem0, sem1)
  semsr = (semr0, semr1)
  semsi = (semi0, semi1, semi2, semi3)

  cid = lax.axis_index("c")
  sid = lax.axis_index("s")
  w = sid * NC + cid

  @pl.when(sid == 0)
  def _():
    pltpu.sync_copy(rel_hbm, rel_sp)

  for q in range(4):
    pltpu.async_copy(idx_hbm.at[w, q], idxs[q], semsi[q])

  def _per_tile_rows(fn):
    @pl.when(sid < NS - 1)
    def _():
      fn(sid * ROWS_PT, ROWS_PT)

    @pl.when(sid == NS - 1)
    def _():
      fn(ROWS_PT * (NS - 1), ROWS_LAST)

  def _zero(base, size):
    pltpu.sync_copy(z_hbm.at[pl.ds(base, size)],
                    agg_sp.at[pl.ds(base, size)])
  _per_tile_rows(_zero)

  for b in range(2):
    pltpu.make_async_copy(idx_hbm.at[w, b], idxs[b], semsi[b]).wait()
    pltpu.async_copy(ent_hbm.at[idxs[b].at[0]], rows[b], sems[b])

  plsc.subcore_barrier()

  for b in range(2):
    pltpu.async_copy(rel_sp.at[idxs[b].at[2]], relr[b], semsr[b])

  def compute_chunk(rows_ref, relr_ref):
    def edge_body(e, _):
      for k in range(8):
        sl = pl.ds(k * 16, 16)
        rows_ref[e, sl] = rows_ref[e, sl] * relr_ref[e, sl]
      return 0
    lax.fori_loop(0, C, edge_body, 0, unroll=2)

  def do_chunk(jb, b, qn, qi):
    pltpu.make_async_copy(ent_hbm.at[idxs[qi].at[0]], rows[b],
                          sems[b]).wait()
    pltpu.make_async_copy(rel_sp.at[idxs[qi].at[2]], relr[b],
                          semsr[b]).wait()
    compute_chunk(rows[b], relr[b])
    pltpu.sync_copy(rows[b], agg_sp.at[idxs[qi].at[1]], add=True)

    @pl.when(jb + 2 < NCH)
    def _():
      pltpu.make_async_copy(idx_hbm.at[w, jb + 2], idxs[qn], semsi[qn]).wait()
      pltpu.async_copy(ent_hbm.at[idxs[qn].at[0]], rows[b], sems[b])
      pltpu.async_copy(rel_sp.at[idxs[qn].at[2]], relr[b], semsr[b])

    @pl.when(jb + 4 < NCH)
    def _():
      pltpu.async_copy(idx_hbm.at[w, jb + 4], idxs[qi], semsi[qi])

  def quad_body(p, _):
    jb = 4 * p
    do_chunk(jb, 0, 2, 0)
    do_chunk(jb + 1, 1, 3, 1)
    do_chunk(jb + 2, 0, 0, 2)
    do_chunk(jb + 3, 1, 1, 3)
    return 0

  lax.fori_loop(0, NCH // 4, quad_body, 0)
  for t in range(NCH - (NCH // 4) * 4):
    jb = (NCH // 4) * 4 + t
    do_chunk(jb, jb % 2, (jb + 2) % 4, jb % 4)

  plsc.subcore_barrier()

  def _writeout(base, size):
    pltpu.sync_copy(agg_sp.at[pl.ds(base, size)],
                    out_hbm.at[cid, pl.ds(base, size)])
  _per_tile_rows(_writeout)


def _make_edge_kernel():
  mesh = plsc.VectorSubcoreMesh(core_axis_name="c", subcore_axis_name="s")
  scratch = (
      [pltpu.VMEM((3, C), jnp.int32) for _ in range(4)]
      + [pltpu.VMEM((C, D), jnp.float32) for _ in range(4)]
      + [pltpu.VMEM_SHARED((N, D), jnp.float32),
         pltpu.VMEM_SHARED((R, D), jnp.float32)]
      + [pltpu.SemaphoreType.DMA] * 8)
  return pl.kernel(
      _edge_kernel_body,
      out_type=[jax.ShapeDtypeStruct((NC, N, D), jnp.float32)],
      mesh=mesh, scratch_types=scratch, name="compgcn_edges")


@functools.lru_cache(maxsize=None)
def _get_edge_kernel():
  return _make_edge_kernel()


def _dense_body(layer, do_relu, p_ref, deg_ref, ent_ref, rel_ref, q_ref,
                w_ref, wl_ref, wr_ref, lr_ref, b_ref, g_ref, be_ref,
                oent_ref, orel_ref):
  pre = p_ref[0] + p_ref[1]
  deg = deg_ref[0, :, 0:1] + deg_ref[1, :, 0:1]
  agg = jnp.dot(pre, w_ref[...], preferred_element_type=jnp.float32)
  agg = agg / jnp.maximum(deg, 1.0)
  loop_msg = jnp.dot(ent_ref[...] * lr_ref[...], wl_ref[...],
                     preferred_element_type=jnp.float32)
  out = 0.5 * (agg + loop_msg) + b_ref[...]
  m = jnp.mean(out, axis=0, keepdims=True)
  v = jnp.mean((out - m) * (out - m), axis=0, keepdims=True)
  out = g_ref[...] * (out - m) * lax.rsqrt(v + 1e-5) + be_ref[...]
  out = jnp.where(q_ref[...] <= layer, ent_ref[...], out)
  if do_relu:
    out = jnp.maximum(out, 0.0)
  oent_ref[...] = out
  orel_ref[...] = jnp.dot(rel_ref[...], wr_ref[...],
                          preferred_element_type=jnp.float32)


def _make_dense(layer, do_relu):
  return pl.pallas_call(
      functools.partial(_dense_body, layer, do_relu),
      out_shape=[jax.ShapeDtypeStruct((N, D), jnp.float32),
                 jax.ShapeDtypeStruct((R, D), jnp.float32)],
      name=f"compgcn_dense_{layer}")


_dense_layers = [_make_dense(0, True), _make_dense(1, True),
                 _make_dense(2, False)]

SEG_ROWS = 320
SEG_GROUPS = SEG_ROWS // 16


def _segmax_body(ent_hbm, bid_hbm, sent_hbm, out_hbm,
                 bid_v, rows_v, stag_v, sent_v):
  cid = lax.axis_index("c")
  sid = lax.axis_index("s")
  w = sid * NC + cid
  base = w * SEG_ROWS
  is_last = w == NW - 1
  mysize = jnp.where(is_last, N - (NW - 1) * SEG_ROWS, SEG_ROWS)
  ngroups = mysize // 16

  pltpu.sync_copy(sent_hbm, sent_v)
  for blk in range(B // 64):
    pltpu.sync_copy(sent_v, out_hbm.at[w, pl.ds(blk * 64, 64)])

  @pl.when(jnp.logical_not(is_last))
  def _():
    pltpu.sync_copy(bid_hbm.at[pl.ds(base, SEG_ROWS)], bid_v)
    pltpu.sync_copy(ent_hbm.at[pl.ds(base, SEG_ROWS)], rows_v)

  @pl.when(is_last)
  def _():
    tail = N - (NW - 1) * SEG_ROWS
    pltpu.sync_copy(bid_hbm.at[pl.ds(base, tail)], bid_v.at[pl.ds(0, tail)])
    pltpu.sync_copy(ent_hbm.at[pl.ds(base, tail)], rows_v.at[pl.ds(0, tail)])

  for k in range(8):
    stag_v[pl.ds(k * 16, 16)] = jnp.full((16,), NEG_SENTINEL, jnp.float32)

  prev0 = bid_v[pl.ds(0, 16)][0]

  def group_scan(g, prev):
    ids16 = bid_v[pl.ds(g * 16, 16)]

    def do_lane(lane, prev_lane):
      rid = ids16[lane]
      changed = jnp.logical_and(rid != prev_lane, g * 16 + lane < mysize)

      @pl.when(changed)
      def _():
        pltpu.sync_copy(stag_v, out_hbm.at[w, prev_lane])

      @pl.when(g * 16 + lane < mysize)
      def _():
        r = g * 16 + lane
        for k in range(8):
          sl = pl.ds(k * 16, 16)
          row = rows_v[r, sl]
          cur = stag_v[sl]
          stag_v[sl] = jnp.where(changed, row, jnp.maximum(cur, row))
      return jnp.where(g * 16 + lane < mysize, rid, prev_lane)

    for lane in range(16):
      prev = do_lane(lane, prev)
    return prev

  prev_last = lax.fori_loop(0, SEG_GROUPS, group_scan, prev0)
  pltpu.sync_copy(stag_v, out_hbm.at[w, prev_last])


@functools.lru_cache(maxsize=None)
def _get_segmax_kernel():
  return pl.kernel(
      _segmax_body,
      out_type=[jax.ShapeDtypeStruct((NW, B, D), jnp.float32)],
      mesh=plsc.VectorSubcoreMesh(core_axis_name="c", subcore_axis_name="s"),
      scratch_types=[
          pltpu.VMEM((SEG_ROWS,), jnp.int32),
          pltpu.VMEM((SEG_ROWS, D), jnp.float32),
          pltpu.VMEM((D,), jnp.float32),
          pltpu.VMEM((64, D), jnp.float32),
      ],
      name="segment_max_partials")


def _segreduce_body(p_ref, out_ref):
  m = jnp.max(p_ref[...], axis=0)
  out_ref[...] = jnp.where(m <= NEG_SENTINEL, 0.0, m)


@functools.lru_cache(maxsize=None)
def _get_segreduce():
  return pl.pallas_call(
      _segreduce_body,
      out_shape=jax.ShapeDtypeStruct((B, D), jnp.float32),
      name="segment_max_reduce")


def kernel(ent_embed, rel_embed, edge_index, edge_type, q_diameters,
           batch_idx, target_idx, W, W_loop, W_rel, loop_rel, bias,
           bn_gamma, bn_beta):
  src = edge_index[0].astype(jnp.int32).reshape(NW, NCH, C)
  dst = edge_index[1].astype(jnp.int32).reshape(NW, NCH, C)
  typ = edge_type.astype(jnp.int32).reshape(NW, NCH, C)
  idx_pack = jnp.stack([src, dst, typ], axis=2)
  z = jnp.zeros((N, D), jnp.float32)
  q2 = q_diameters.astype(jnp.int32).reshape(N, 1)
  lr = loop_rel.reshape(1, D)
  b2 = bias.reshape(1, D)
  g2 = bn_gamma.reshape(1, D)
  be2 = bn_beta.reshape(1, D)

  ent = ent_embed
  rel = rel_embed
  (deg16,) = _get_edge_kernel()(jnp.ones((N, D), jnp.float32),
                                jnp.ones((R, D), jnp.float32), idx_pack, z)
  for layer in range(3):
    (parts,) = _get_edge_kernel()(ent, rel, idx_pack, z)
    ent, rel = _dense_layers[layer](parts, deg16, ent, rel, q2,
                                    W, W_loop, W_rel, lr, b2, g2, be2)

  sent = jnp.full((64, D), NEG_SENTINEL, jnp.float32)
  (partials,) = _get_segmax_kernel()(ent, batch_idx.astype(jnp.int32), sent)
  return _get_segreduce()(partials)


# speedup vs baseline: 2.0708x; 2.0310x over previous
import functools

import jax
import jax.numpy as jnp
from jax import lax
from jax.experimental import pallas as pl
from jax.experimental.pallas import tpu as pltpu
from jax.experimental.pallas import tpu_sc as plsc

N = 10000
E = 320000
D = 128
R = 500
B = 1024

NC = 2
NS = 16
NW = NC * NS
EPT = E // NW
C = 80
NCH = EPT // C
ROWS_PT = 624
ROWS_LAST = N - ROWS_PT * (NS - 1)

NEG_SENTINEL = -3.0e38


def _edge_kernel_body(ent_hbm, rel_hbm, idx_hbm, z_hbm, out_hbm,
                      idx0, idx1, idx2, idx3, rows0, rows1, relr0, relr1,
                      agg_sp, rel_sp,
                      sem0, sem1, semr0, semr1, semi0, semi1, semi2, semi3):
  idxs = (idx0, idx1, idx2, idx3)
  rows = (rows0, rows1)
  relr = (relr0, relr1)
  sems = (sem0, sem1)
  semsr = (semr0, semr1)
  semsi = (semi0, semi1, semi2, semi3)

  cid = lax.axis_index("c")
  sid = lax.axis_index("s")
  w = sid * NC + cid

  @pl.when(sid == 0)
  def _():
    pltpu.sync_copy(rel_hbm, rel_sp)

  for q in range(4):
    pltpu.async_copy(idx_hbm.at[w, q], idxs[q], semsi[q])

  def _per_tile_rows(fn):
    @pl.when(sid < NS - 1)
    def _():
      fn(sid * ROWS_PT, ROWS_PT)

    @pl.when(sid == NS - 1)
    def _():
      fn(ROWS_PT * (NS - 1), ROWS_LAST)

  def _zero(base, size):
    pltpu.sync_copy(z_hbm.at[pl.ds(base, size)],
                    agg_sp.at[pl.ds(base, size)])
  _per_tile_rows(_zero)

  for b in range(2):
    pltpu.make_async_copy(idx_hbm.at[w, b], idxs[b], semsi[b]).wait()
    pltpu.async_copy(ent_hbm.at[idxs[b].at[0]], rows[b], sems[b])

  plsc.subcore_barrier()

  for b in range(2):
    pltpu.async_copy(rel_sp.at[idxs[b].at[2]], relr[b], semsr[b])

  def compute_chunk(rows_ref, relr_ref):
    def edge_body(e, _):
      for k in range(8):
        sl = pl.ds(k * 16, 16)
        rows_ref[e, sl] = rows_ref[e, sl] * relr_ref[e, sl]
      return 0
    lax.fori_loop(0, C, edge_body, 0)

  def do_chunk(jb, b, qn, qi):
    pltpu.make_async_copy(ent_hbm.at[idxs[qi].at[0]], rows[b],
                          sems[b]).wait()
    pltpu.make_async_copy(rel_sp.at[idxs[qi].at[2]], relr[b],
                          semsr[b]).wait()
    compute_chunk(rows[b], relr[b])
    pltpu.sync_copy(rows[b], agg_sp.at[idxs[qi].at[1]], add=True)

    @pl.when(jb + 2 < NCH)
    def _():
      pltpu.make_async_copy(idx_hbm.at[w, jb + 2], idxs[qn], semsi[qn]).wait()
      pltpu.async_copy(ent_hbm.at[idxs[qn].at[0]], rows[b], sems[b])
      pltpu.async_copy(rel_sp.at[idxs[qn].at[2]], relr[b], semsr[b])

    @pl.when(jb + 4 < NCH)
    def _():
      pltpu.async_copy(idx_hbm.at[w, jb + 4], idxs[qi], semsi[qi])

  def quad_body(p, _):
    jb = 4 * p
    do_chunk(jb, 0, 2, 0)
    do_chunk(jb + 1, 1, 3, 1)
    do_chunk(jb + 2, 0, 0, 2)
    do_chunk(jb + 3, 1, 1, 3)
    return 0

  lax.fori_loop(0, NCH // 4, quad_body, 0)
  for t in range(NCH - (NCH // 4) * 4):
    jb = (NCH // 4) * 4 + t
    do_chunk(jb, jb % 2, (jb + 2) % 4, jb % 4)

  plsc.subcore_barrier()

  def _writeout(base, size):
    pltpu.sync_copy(agg_sp.at[pl.ds(base, size)],
                    out_hbm.at[cid, pl.ds(base, size)])
  _per_tile_rows(_writeout)


def _make_edge_kernel():
  mesh = plsc.VectorSubcoreMesh(core_axis_name="c", subcore_axis_name="s")
  scratch = (
      [pltpu.VMEM((3, C), jnp.int32) for _ in range(4)]
      + [pltpu.VMEM((C, D), jnp.float32) for _ in range(4)]
      + [pltpu.VMEM_SHARED((N, D), jnp.float32),
         pltpu.VMEM_SHARED((R, D), jnp.float32)]
      + [pltpu.SemaphoreType.DMA] * 8)
  return pl.kernel(
      _edge_kernel_body,
      out_type=[jax.ShapeDtypeStruct((NC, N, D), jnp.float32)],
      mesh=mesh, scratch_types=scratch, name="compgcn_edges")


@functools.lru_cache(maxsize=None)
def _get_edge_kernel():
  return _make_edge_kernel()


def _dense_body(layer, do_relu, p_ref, deg_ref, ent_ref, rel_ref, q_ref,
                w_ref, wl_ref, wr_ref, lr_ref, b_ref, g_ref, be_ref,
                oent_ref, orel_ref):
  pre = p_ref[0] + p_ref[1]
  deg = deg_ref[0, :, 0:1] + deg_ref[1, :, 0:1]
  agg = jnp.dot(pre, w_ref[...], preferred_element_type=jnp.float32)
  agg = agg / jnp.maximum(deg, 1.0)
  loop_msg = jnp.dot(ent_ref[...] * lr_ref[...], wl_ref[...],
                     preferred_element_type=jnp.float32)
  out = 0.5 * (agg + loop_msg) + b_ref[...]
  m = jnp.mean(out, axis=0, keepdims=True)
  v = jnp.mean((out - m) * (out - m), axis=0, keepdims=True)
  out = g_ref[...] * (out - m) * lax.rsqrt(v + 1e-5) + be_ref[...]
  out = jnp.where(q_ref[...] <= layer, ent_ref[...], out)
  if do_relu:
    out = jnp.maximum(out, 0.0)
  oent_ref[...] = out
  orel_ref[...] = jnp.dot(rel_ref[...], wr_ref[...],
                          preferred_element_type=jnp.float32)


def _make_dense(layer, do_relu):
  return pl.pallas_call(
      functools.partial(_dense_body, layer, do_relu),
      out_shape=[jax.ShapeDtypeStruct((N, D), jnp.float32),
                 jax.ShapeDtypeStruct((R, D), jnp.float32)],
      name=f"compgcn_dense_{layer}")


_dense_layers = [_make_dense(0, True), _make_dense(1, True),
                 _make_dense(2, False)]

SEG_ROWS = 320
SEG_GROUPS = SEG_ROWS // 16


def _segmax_body(ent_hbm, bid_hbm, sent_hbm, out_hbm,
                 bid_v, rows_v, stag_v, sent_v):
  cid = lax.axis_index("c")
  sid = lax.axis_index("s")
  w = sid * NC + cid
  base = w * SEG_ROWS
  is_last = w == NW - 1
  mysize = jnp.where(is_last, N - (NW - 1) * SEG_ROWS, SEG_ROWS)
  ngroups = mysize // 16

  pltpu.sync_copy(sent_hbm, sent_v)
  for blk in range(B // 64):
    pltpu.sync_copy(sent_v, out_hbm.at[w, pl.ds(blk * 64, 64)])

  @pl.when(jnp.logical_not(is_last))
  def _():
    pltpu.sync_copy(bid_hbm.at[pl.ds(base, SEG_ROWS)], bid_v)
    pltpu.sync_copy(ent_hbm.at[pl.ds(base, SEG_ROWS)], rows_v)

  @pl.when(is_last)
  def _():
    tail = N - (NW - 1) * SEG_ROWS
    pltpu.sync_copy(bid_hbm.at[pl.ds(base, tail)], bid_v.at[pl.ds(0, tail)])
    pltpu.sync_copy(ent_hbm.at[pl.ds(base, tail)], rows_v.at[pl.ds(0, tail)])

  for k in range(8):
    stag_v[pl.ds(k * 16, 16)] = jnp.full((16,), NEG_SENTINEL, jnp.float32)

  prev0 = bid_v[pl.ds(0, 16)][0]

  def group_scan(g, prev):
    ids16 = bid_v[pl.ds(g * 16, 16)]

    def do_lane(lane, prev_lane):
      rid = ids16[lane]
      changed = jnp.logical_and(rid != prev_lane, g * 16 + lane < mysize)

      @pl.when(changed)
      def _():
        pltpu.sync_copy(stag_v, out_hbm.at[w, prev_lane])

      @pl.when(g * 16 + lane < mysize)
      def _():
        r = g * 16 + lane
        for k in range(8):
          sl = pl.ds(k * 16, 16)
          row = rows_v[r, sl]
          cur = stag_v[sl]
          stag_v[sl] = jnp.where(changed, row, jnp.maximum(cur, row))
      return jnp.where(g * 16 + lane < mysize, rid, prev_lane)

    for lane in range(16):
      prev = do_lane(lane, prev)
    return prev

  prev_last = lax.fori_loop(0, SEG_GROUPS, group_scan, prev0)
  pltpu.sync_copy(stag_v, out_hbm.at[w, prev_last])


@functools.lru_cache(maxsize=None)
def _get_segmax_kernel():
  return pl.kernel(
      _segmax_body,
      out_type=[jax.ShapeDtypeStruct((NW, B, D), jnp.float32)],
      mesh=plsc.VectorSubcoreMesh(core_axis_name="c", subcore_axis_name="s"),
      scratch_types=[
          pltpu.VMEM((SEG_ROWS,), jnp.int32),
          pltpu.VMEM((SEG_ROWS, D), jnp.float32),
          pltpu.VMEM((D,), jnp.float32),
          pltpu.VMEM((64, D), jnp.float32),
      ],
      name="segment_max_partials")


def _segreduce_body(p_ref, out_ref):
  m = jnp.max(p_ref[...], axis=0)
  out_ref[...] = jnp.where(m <= NEG_SENTINEL, 0.0, m)


@functools.lru_cache(maxsize=None)
def _get_segreduce():
  return pl.pallas_call(
      _segreduce_body,
      out_shape=jax.ShapeDtypeStruct((B, D), jnp.float32),
      name="segment_max_reduce")


def kernel(ent_embed, rel_embed, edge_index, edge_type, q_diameters,
           batch_idx, target_idx, W, W_loop, W_rel, loop_rel, bias,
           bn_gamma, bn_beta):
  src = edge_index[0].astype(jnp.int32).reshape(NW, NCH, C)
  dst = edge_index[1].astype(jnp.int32).reshape(NW, NCH, C)
  typ = edge_type.astype(jnp.int32).reshape(NW, NCH, C)
  idx_pack = jnp.stack([src, dst, typ], axis=2)
  z = jnp.zeros((N, D), jnp.float32)
  q2 = q_diameters.astype(jnp.int32).reshape(N, 1)
  lr = loop_rel.reshape(1, D)
  b2 = bias.reshape(1, D)
  g2 = bn_gamma.reshape(1, D)
  be2 = bn_beta.reshape(1, D)

  ent = ent_embed
  rel = rel_embed
  (deg16,) = _get_edge_kernel()(jnp.ones((N, D), jnp.float32),
                                jnp.ones((R, D), jnp.float32), idx_pack, z)
  for layer in range(3):
    (parts,) = _get_edge_kernel()(ent, rel, idx_pack, z)
    ent, rel = _dense_layers[layer](parts, deg16, ent, rel, q2,
                                    W, W_loop, W_rel, lr, b2, g2, be2)

  sent = jnp.full((64, D), NEG_SENTINEL, jnp.float32)
  (partials,) = _get_segmax_kernel()(ent, batch_idx.astype(jnp.int32), sent)
  return _get_segreduce()(partials)


# speedup vs baseline: 2.1007x; 1.0144x over previous
import functools

import jax
import jax.numpy as jnp
from jax import lax
from jax.experimental import pallas as pl
from jax.experimental.pallas import tpu as pltpu
from jax.experimental.pallas import tpu_sc as plsc

N = 10000
E = 320000
D = 128
R = 500
B = 1024

NC = 2
NS = 16
NW = NC * NS
EPT = E // NW
C = 80
NCH = EPT // C
ROWS_PT = 624
ROWS_LAST = N - ROWS_PT * (NS - 1)

NEG_SENTINEL = -3.0e38


def _edge_kernel_body(ent_hbm, rel_hbm, idx_hbm, z_hbm, out_hbm,
                      idx0, idx1, idx2, idx3, rows0, rows1, relr0, relr1,
                      agg_sp, rel_sp,
                      sem0, sem1, semr0, semr1, semi0, semi1, semi2, semi3):
  idxs = (idx0, idx1, idx2, idx3)
  rows = (rows0, rows1)
  relr = (relr0, relr1)
  sems = (sem0, sem1)
  semsr = (semr0, semr1)
  semsi = (semi0, semi1, semi2, semi3)

  cid = lax.axis_index("c")
  sid = lax.axis_index("s")
  w = sid * NC + cid

  @pl.when(sid == 0)
  def _():
    pltpu.sync_copy(rel_hbm, rel_sp)

  for q in range(4):
    pltpu.async_copy(idx_hbm.at[w, q], idxs[q], semsi[q])

  def _per_tile_rows(fn):
    @pl.when(sid < NS - 1)
    def _():
      fn(sid * ROWS_PT, ROWS_PT)

    @pl.when(sid == NS - 1)
    def _():
      fn(ROWS_PT * (NS - 1), ROWS_LAST)

  def _zero(base, size):
    pltpu.sync_copy(z_hbm.at[pl.ds(base, size)],
                    agg_sp.at[pl.ds(base, size)])
  _per_tile_rows(_zero)

  for b in range(2):
    pltpu.make_async_copy(idx_hbm.at[w, b], idxs[b], semsi[b]).wait()
    pltpu.async_copy(ent_hbm.at[idxs[b].at[0]], rows[b], sems[b])

  plsc.subcore_barrier()

  for b in range(2):
    pltpu.async_copy(rel_sp.at[idxs[b].at[2]], relr[b], semsr[b])

  def compute_chunk(rows_ref, relr_ref):
    @plsc.parallel_loop(0, C, 1, unroll=4)
    def _(e):
      for k in range(8):
        sl = pl.ds(k * 16, 16)
        rows_ref[e, sl] = rows_ref[e, sl] * relr_ref[e, sl]

  def do_chunk(jb, b, qn, qi):
    pltpu.make_async_copy(ent_hbm.at[idxs[qi].at[0]], rows[b],
                          sems[b]).wait()
    pltpu.make_async_copy(rel_sp.at[idxs[qi].at[2]], relr[b],
                          semsr[b]).wait()
    compute_chunk(rows[b], relr[b])
    pltpu.sync_copy(rows[b], agg_sp.at[idxs[qi].at[1]], add=True)

    @pl.when(jb + 2 < NCH)
    def _():
      pltpu.make_async_copy(idx_hbm.at[w, jb + 2], idxs[qn], semsi[qn]).wait()
      pltpu.async_copy(ent_hbm.at[idxs[qn].at[0]], rows[b], sems[b])
      pltpu.async_copy(rel_sp.at[idxs[qn].at[2]], relr[b], semsr[b])

    @pl.when(jb + 4 < NCH)
    def _():
      pltpu.async_copy(idx_hbm.at[w, jb + 4], idxs[qi], semsi[qi])

  def quad_body(p, _):
    jb = 4 * p
    do_chunk(jb, 0, 2, 0)
    do_chunk(jb + 1, 1, 3, 1)
    do_chunk(jb + 2, 0, 0, 2)
    do_chunk(jb + 3, 1, 1, 3)
    return 0

  lax.fori_loop(0, NCH // 4, quad_body, 0)
  for t in range(NCH - (NCH // 4) * 4):
    jb = (NCH // 4) * 4 + t
    do_chunk(jb, jb % 2, (jb + 2) % 4, jb % 4)

  plsc.subcore_barrier()

  def _writeout(base, size):
    pltpu.sync_copy(agg_sp.at[pl.ds(base, size)],
                    out_hbm.at[cid, pl.ds(base, size)])
  _per_tile_rows(_writeout)


def _make_edge_kernel():
  mesh = plsc.VectorSubcoreMesh(core_axis_name="c", subcore_axis_name="s")
  scratch = (
      [pltpu.VMEM((3, C), jnp.int32) for _ in range(4)]
      + [pltpu.VMEM((C, D), jnp.float32) for _ in range(4)]
      + [pltpu.VMEM_SHARED((N, D), jnp.float32),
         pltpu.VMEM_SHARED((R, D), jnp.float32)]
      + [pltpu.SemaphoreType.DMA] * 8)
  return pl.kernel(
      _edge_kernel_body,
      out_type=[jax.ShapeDtypeStruct((NC, N, D), jnp.float32)],
      mesh=mesh, scratch_types=scratch, name="compgcn_edges")


@functools.lru_cache(maxsize=None)
def _get_edge_kernel():
  return _make_edge_kernel()


def _dense_body(layer, do_relu, p_ref, deg_ref, ent_ref, rel_ref, q_ref,
                w_ref, wl_ref, wr_ref, lr_ref, b_ref, g_ref, be_ref,
                oent_ref, orel_ref):
  pre = p_ref[0] + p_ref[1]
  deg = deg_ref[0, :, 0:1] + deg_ref[1, :, 0:1]
  agg = jnp.dot(pre, w_ref[...], preferred_element_type=jnp.float32)
  agg = agg / jnp.maximum(deg, 1.0)
  loop_msg = jnp.dot(ent_ref[...] * lr_ref[...], wl_ref[...],
                     preferred_element_type=jnp.float32)
  out = 0.5 * (agg + loop_msg) + b_ref[...]
  m = jnp.mean(out, axis=0, keepdims=True)
  v = jnp.mean((out - m) * (out - m), axis=0, keepdims=True)
  out = g_ref[...] * (out - m) * lax.rsqrt(v + 1e-5) + be_ref[...]
  out = jnp.where(q_ref[...] <= layer, ent_ref[...], out)
  if do_relu:
    out = jnp.maximum(out, 0.0)
  oent_ref[...] = out
  orel_ref[...] = jnp.dot(rel_ref[...], wr_ref[...],
                          preferred_element_type=jnp.float32)


def _make_dense(layer, do_relu):
  return pl.pallas_call(
      functools.partial(_dense_body, layer, do_relu),
      out_shape=[jax.ShapeDtypeStruct((N, D), jnp.float32),
                 jax.ShapeDtypeStruct((R, D), jnp.float32)],
      name=f"compgcn_dense_{layer}")


_dense_layers = [_make_dense(0, True), _make_dense(1, True),
                 _make_dense(2, False)]

SEG_ROWS = 320
SEG_GROUPS = SEG_ROWS // 16


def _segmax_body(ent_hbm, bid_hbm, sent_hbm, out_hbm,
                 bid_v, rows_v, stag_v, sent_v):
  cid = lax.axis_index("c")
  sid = lax.axis_index("s")
  w = sid * NC + cid
  base = w * SEG_ROWS
  is_last = w == NW - 1
  mysize = jnp.where(is_last, N - (NW - 1) * SEG_ROWS, SEG_ROWS)
  ngroups = mysize // 16

  pltpu.sync_copy(sent_hbm, sent_v)
  for blk in range(B // 64):
    pltpu.sync_copy(sent_v, out_hbm.at[w, pl.ds(blk * 64, 64)])

  @pl.when(jnp.logical_not(is_last))
  def _():
    pltpu.sync_copy(bid_hbm.at[pl.ds(base, SEG_ROWS)], bid_v)
    pltpu.sync_copy(ent_hbm.at[pl.ds(base, SEG_ROWS)], rows_v)

  @pl.when(is_last)
  def _():
    tail = N - (NW - 1) * SEG_ROWS
    pltpu.sync_copy(bid_hbm.at[pl.ds(base, tail)], bid_v.at[pl.ds(0, tail)])
    pltpu.sync_copy(ent_hbm.at[pl.ds(base, tail)], rows_v.at[pl.ds(0, tail)])

  for k in range(8):
    stag_v[pl.ds(k * 16, 16)] = jnp.full((16,), NEG_SENTINEL, jnp.float32)

  prev0 = bid_v[pl.ds(0, 16)][0]

  def group_scan(g, prev):
    ids16 = bid_v[pl.ds(g * 16, 16)]

    def do_lane(lane, prev_lane):
      rid = ids16[lane]
      changed = jnp.logical_and(rid != prev_lane, g * 16 + lane < mysize)

      @pl.when(changed)
      def _():
        pltpu.sync_copy(stag_v, out_hbm.at[w, prev_lane])

      @pl.when(g * 16 + lane < mysize)
      def _():
        r = g * 16 + lane
        for k in range(8):
          sl = pl.ds(k * 16, 16)
          row = rows_v[r, sl]
          cur = stag_v[sl]
          stag_v[sl] = jnp.where(changed, row, jnp.maximum(cur, row))
      return jnp.where(g * 16 + lane < mysize, rid, prev_lane)

    for lane in range(16):
      prev = do_lane(lane, prev)
    return prev

  prev_last = lax.fori_loop(0, SEG_GROUPS, group_scan, prev0)
  pltpu.sync_copy(stag_v, out_hbm.at[w, prev_last])


@functools.lru_cache(maxsize=None)
def _get_segmax_kernel():
  return pl.kernel(
      _segmax_body,
      out_type=[jax.ShapeDtypeStruct((NW, B, D), jnp.float32)],
      mesh=plsc.VectorSubcoreMesh(core_axis_name="c", subcore_axis_name="s"),
      scratch_types=[
          pltpu.VMEM((SEG_ROWS,), jnp.int32),
          pltpu.VMEM((SEG_ROWS, D), jnp.float32),
          pltpu.VMEM((D,), jnp.float32),
          pltpu.VMEM((64, D), jnp.float32),
      ],
      name="segment_max_partials")


def _segreduce_body(p_ref, out_ref):
  m = jnp.max(p_ref[...], axis=0)
  out_ref[...] = jnp.where(m <= NEG_SENTINEL, 0.0, m)


@functools.lru_cache(maxsize=None)
def _get_segreduce():
  return pl.pallas_call(
      _segreduce_body,
      out_shape=jax.ShapeDtypeStruct((B, D), jnp.float32),
      name="segment_max_reduce")


def kernel(ent_embed, rel_embed, edge_index, edge_type, q_diameters,
           batch_idx, target_idx, W, W_loop, W_rel, loop_rel, bias,
           bn_gamma, bn_beta):
  src = edge_index[0].astype(jnp.int32).reshape(NW, NCH, C)
  dst = edge_index[1].astype(jnp.int32).reshape(NW, NCH, C)
  typ = edge_type.astype(jnp.int32).reshape(NW, NCH, C)
  idx_pack = jnp.stack([src, dst, typ], axis=2)
  z = jnp.zeros((N, D), jnp.float32)
  q2 = q_diameters.astype(jnp.int32).reshape(N, 1)
  lr = loop_rel.reshape(1, D)
  b2 = bias.reshape(1, D)
  g2 = bn_gamma.reshape(1, D)
  be2 = bn_beta.reshape(1, D)

  ent = ent_embed
  rel = rel_embed
  (deg16,) = _get_edge_kernel()(jnp.ones((N, D), jnp.float32),
                                jnp.ones((R, D), jnp.float32), idx_pack, z)
  for layer in range(3):
    (parts,) = _get_edge_kernel()(ent, rel, idx_pack, z)
    ent, rel = _dense_layers[layer](parts, deg16, ent, rel, q2,
                                    W, W_loop, W_rel, lr, b2, g2, be2)

  sent = jnp.full((64, D), NEG_SENTINEL, jnp.float32)
  (partials,) = _get_segmax_kernel()(ent, batch_idx.astype(jnp.int32), sent)
  return _get_segreduce()(partials)


# speedup vs baseline: 2.4236x; 1.1537x over previous
import functools

import jax
import jax.numpy as jnp
from jax import lax
from jax.experimental import pallas as pl
from jax.experimental.pallas import tpu as pltpu
from jax.experimental.pallas import tpu_sc as plsc

N = 10000
E = 320000
D = 128
R = 500
B = 1024

NC = 2
NS = 16
NW = NC * NS
EPT = E // NW
C = 80
NCH = EPT // C
ROWS_PT = 624
ROWS_LAST = N - ROWS_PT * (NS - 1)

NEG_SENTINEL = -3.0e38


def _edge_kernel_body(ent_hbm, rel_hbm, idx_hbm, z_hbm, out_hbm,
                      idx0, idx1, idx2, idx3, rows0, rows1, relr0, relr1,
                      agg_sp, rel_sp,
                      sem0, sem1, semr0, semr1, semi0, semi1, semi2, semi3):
  idxs = (idx0, idx1, idx2, idx3)
  rows = (rows0, rows1)
  relr = (relr0, relr1)
  sems = (sem0, sem1)
  semsr = (semr0, semr1)
  semsi = (semi0, semi1, semi2, semi3)

  cid = lax.axis_index("c")
  sid = lax.axis_index("s")
  w = sid * NC + cid

  @pl.when(sid == 0)
  def _():
    pltpu.sync_copy(rel_hbm, rel_sp)

  for q in range(4):
    pltpu.async_copy(idx_hbm.at[w, q], idxs[q], semsi[q])

  def _per_tile_rows(fn):
    @pl.when(sid < NS - 1)
    def _():
      fn(sid * ROWS_PT, ROWS_PT)

    @pl.when(sid == NS - 1)
    def _():
      fn(ROWS_PT * (NS - 1), ROWS_LAST)

  def _zero(base, size):
    pltpu.sync_copy(z_hbm.at[pl.ds(base, size)],
                    agg_sp.at[pl.ds(base, size)])
  _per_tile_rows(_zero)

  for b in range(2):
    pltpu.make_async_copy(idx_hbm.at[w, b], idxs[b], semsi[b]).wait()
    pltpu.async_copy(ent_hbm.at[idxs[b].at[0]], rows[b], sems[b])

  plsc.subcore_barrier()

  for b in range(2):
    pltpu.async_copy(rel_sp.at[idxs[b].at[2]], relr[b], semsr[b])

  def compute_chunk(rows_ref, relr_ref):
    @plsc.parallel_loop(0, C, 1, unroll=4)
    def _(e):
      for k in range(8):
        sl = pl.ds(k * 16, 16)
        rows_ref[e, sl] = rows_ref[e, sl] * relr_ref[e, sl]

  def do_chunk(jb, b, qn, qi):
    pltpu.make_async_copy(ent_hbm.at[idxs[qi].at[0]], rows[b],
                          sems[b]).wait()
    pltpu.make_async_copy(rel_sp.at[idxs[qi].at[2]], relr[b],
                          semsr[b]).wait()
    compute_chunk(rows[b], relr[b])
    pltpu.sync_copy(rows[b], agg_sp.at[idxs[qi].at[1]], add=True)

    @pl.when(jb + 2 < NCH)
    def _():
      pltpu.make_async_copy(idx_hbm.at[w, jb + 2], idxs[qn], semsi[qn]).wait()
      pltpu.async_copy(ent_hbm.at[idxs[qn].at[0]], rows[b], sems[b])
      pltpu.async_copy(rel_sp.at[idxs[qn].at[2]], relr[b], semsr[b])

    @pl.when(jb + 4 < NCH)
    def _():
      pltpu.async_copy(idx_hbm.at[w, jb + 4], idxs[qi], semsi[qi])

  def quad_body(p, _):
    jb = 4 * p
    do_chunk(jb, 0, 2, 0)
    do_chunk(jb + 1, 1, 3, 1)
    do_chunk(jb + 2, 0, 0, 2)
    do_chunk(jb + 3, 1, 1, 3)
    return 0

  lax.fori_loop(0, NCH // 4, quad_body, 0)
  for t in range(NCH - (NCH // 4) * 4):
    jb = (NCH // 4) * 4 + t
    do_chunk(jb, jb % 2, (jb + 2) % 4, jb % 4)

  plsc.subcore_barrier()

  def _writeout(base, size):
    pltpu.sync_copy(agg_sp.at[pl.ds(base, size)],
                    out_hbm.at[cid, pl.ds(base, size)])
  _per_tile_rows(_writeout)


def _make_edge_kernel():
  mesh = plsc.VectorSubcoreMesh(core_axis_name="c", subcore_axis_name="s")
  scratch = (
      [pltpu.VMEM((3, C), jnp.int32) for _ in range(4)]
      + [pltpu.VMEM((C, D), jnp.float32) for _ in range(4)]
      + [pltpu.VMEM_SHARED((N, D), jnp.float32),
         pltpu.VMEM_SHARED((R, D), jnp.float32)]
      + [pltpu.SemaphoreType.DMA] * 8)
  return pl.kernel(
      _edge_kernel_body,
      out_type=[jax.ShapeDtypeStruct((NC, N, D), jnp.float32)],
      mesh=mesh, scratch_types=scratch, name="compgcn_edges")


def _deg_kernel_body(dst_hbm, z_hbm, deg_hbm, dst_v, ones_v, deg_sp,
                     *sems_s):
  cid = lax.axis_index("c")
  sid = lax.axis_index("s")
  w = sid * NC + cid

  pltpu.sync_copy(dst_hbm.at[w], dst_v)

  def fill_ones(i, _):
    for k in range(8):
      ones_v[i, pl.ds(k * 16, 16)] = jnp.full((16,), 1.0, jnp.float32)
    return 0
  lax.fori_loop(0, C, fill_ones, 0)

  def _per_tile_rows(fn):
    @pl.when(sid < NS - 1)
    def _():
      fn(sid * ROWS_PT, ROWS_PT)

    @pl.when(sid == NS - 1)
    def _():
      fn(ROWS_PT * (NS - 1), ROWS_LAST)

  def _zero(base, size):
    pltpu.sync_copy(z_hbm.at[pl.ds(base, size)],
                    deg_sp.at[pl.ds(base, size)])
  _per_tile_rows(_zero)
  plsc.subcore_barrier()

  NSEM = len(sems_s)

  def do_scatter(jb, q):
    @pl.when(jb >= NSEM)
    def _():
      pltpu.make_async_copy(ones_v, deg_sp.at[dst_v.at[jb - NSEM]],
                            sems_s[q]).wait()
    pltpu.async_copy(ones_v, deg_sp.at[dst_v.at[jb]], sems_s[q], add=True)

  def oct_body(p, _):
    for t in range(NSEM):
      do_scatter(NSEM * p + t, t)
    return 0

  lax.fori_loop(0, NCH // NSEM, oct_body, 0)
  for t in range(NCH - (NCH // NSEM) * NSEM):
    do_scatter((NCH // NSEM) * NSEM + t, t)
  for jb in range(NCH - NSEM, NCH):
    pltpu.make_async_copy(ones_v, deg_sp.at[dst_v.at[jb]],
                          sems_s[jb % NSEM]).wait()

  plsc.subcore_barrier()

  def _writeout(base, size):
    pltpu.sync_copy(deg_sp.at[pl.ds(base, size)],
                    deg_hbm.at[cid, pl.ds(base, size)])
  _per_tile_rows(_writeout)


@functools.lru_cache(maxsize=None)
def _get_deg_kernel():
  mesh = plsc.VectorSubcoreMesh(core_axis_name="c", subcore_axis_name="s")
  return pl.kernel(
      _deg_kernel_body,
      out_type=[jax.ShapeDtypeStruct((NC, N, D), jnp.float32)],
      mesh=mesh,
      scratch_types=[
          pltpu.VMEM((NCH, C), jnp.int32),
          pltpu.VMEM((C, D), jnp.float32),
          pltpu.VMEM_SHARED((N, D), jnp.float32),
      ] + [pltpu.SemaphoreType.DMA] * 8,
      name="compgcn_deg")


@functools.lru_cache(maxsize=None)
def _get_edge_kernel():
  return _make_edge_kernel()


def _dense_body(layer, do_relu, p_ref, deg_ref, ent_ref, rel_ref, q_ref,
                w_ref, wl_ref, wr_ref, lr_ref, b_ref, g_ref, be_ref,
                oent_ref, orel_ref):
  pre = p_ref[0] + p_ref[1]
  deg = deg_ref[0, :, 0:1] + deg_ref[1, :, 0:1]
  agg = jnp.dot(pre, w_ref[...], preferred_element_type=jnp.float32)
  agg = agg / jnp.maximum(deg, 1.0)
  loop_msg = jnp.dot(ent_ref[...] * lr_ref[...], wl_ref[...],
                     preferred_element_type=jnp.float32)
  out = 0.5 * (agg + loop_msg) + b_ref[...]
  m = jnp.mean(out, axis=0, keepdims=True)
  v = jnp.mean((out - m) * (out - m), axis=0, keepdims=True)
  out = g_ref[...] * (out - m) * lax.rsqrt(v + 1e-5) + be_ref[...]
  out = jnp.where(q_ref[...] <= layer, ent_ref[...], out)
  if do_relu:
    out = jnp.maximum(out, 0.0)
  oent_ref[...] = out
  orel_ref[...] = jnp.dot(rel_ref[...], wr_ref[...],
                          preferred_element_type=jnp.float32)


def _make_dense(layer, do_relu):
  return pl.pallas_call(
      functools.partial(_dense_body, layer, do_relu),
      out_shape=[jax.ShapeDtypeStruct((N, D), jnp.float32),
                 jax.ShapeDtypeStruct((R, D), jnp.float32)],
      name=f"compgcn_dense_{layer}")


_dense_layers = [_make_dense(0, True), _make_dense(1, True),
                 _make_dense(2, False)]

SEG_ROWS = 320
SEG_GROUPS = SEG_ROWS // 16


def _segmax_body(ent_hbm, bid_hbm, sent_hbm, out_hbm,
                 bid_v, rows_v, stag_v, sent_v):
  cid = lax.axis_index("c")
  sid = lax.axis_index("s")
  w = sid * NC + cid
  base = w * SEG_ROWS
  is_last = w == NW - 1
  mysize = jnp.where(is_last, N - (NW - 1) * SEG_ROWS, SEG_ROWS)
  ngroups = mysize // 16

  pltpu.sync_copy(sent_hbm, sent_v)
  for blk in range(B // 64):
    pltpu.sync_copy(sent_v, out_hbm.at[w, pl.ds(blk * 64, 64)])

  @pl.when(jnp.logical_not(is_last))
  def _():
    pltpu.sync_copy(bid_hbm.at[pl.ds(base, SEG_ROWS)], bid_v)
    pltpu.sync_copy(ent_hbm.at[pl.ds(base, SEG_ROWS)], rows_v)

  @pl.when(is_last)
  def _():
    tail = N - (NW - 1) * SEG_ROWS
    pltpu.sync_copy(bid_hbm.at[pl.ds(base, tail)], bid_v.at[pl.ds(0, tail)])
    pltpu.sync_copy(ent_hbm.at[pl.ds(base, tail)], rows_v.at[pl.ds(0, tail)])

  for k in range(8):
    stag_v[pl.ds(k * 16, 16)] = jnp.full((16,), NEG_SENTINEL, jnp.float32)

  prev0 = bid_v[pl.ds(0, 16)][0]

  def group_scan(g, prev):
    ids16 = bid_v[pl.ds(g * 16, 16)]

    def do_lane(lane, prev_lane):
      rid = ids16[lane]
      changed = jnp.logical_and(rid != prev_lane, g * 16 + lane < mysize)

      @pl.when(changed)
      def _():
        pltpu.sync_copy(stag_v, out_hbm.at[w, prev_lane])

      @pl.when(g * 16 + lane < mysize)
      def _():
        r = g * 16 + lane
        for k in range(8):
          sl = pl.ds(k * 16, 16)
          row = rows_v[r, sl]
          cur = stag_v[sl]
          stag_v[sl] = jnp.where(changed, row, jnp.maximum(cur, row))
      return jnp.where(g * 16 + lane < mysize, rid, prev_lane)

    for lane in range(16):
      prev = do_lane(lane, prev)
    return prev

  prev_last = lax.fori_loop(0, SEG_GROUPS, group_scan, prev0)
  pltpu.sync_copy(stag_v, out_hbm.at[w, prev_last])


@functools.lru_cache(maxsize=None)
def _get_segmax_kernel():
  return pl.kernel(
      _segmax_body,
      out_type=[jax.ShapeDtypeStruct((NW, B, D), jnp.float32)],
      mesh=plsc.VectorSubcoreMesh(core_axis_name="c", subcore_axis_name="s"),
      scratch_types=[
          pltpu.VMEM((SEG_ROWS,), jnp.int32),
          pltpu.VMEM((SEG_ROWS, D), jnp.float32),
          pltpu.VMEM((D,), jnp.float32),
          pltpu.VMEM((64, D), jnp.float32),
      ],
      name="segment_max_partials")


def _segreduce_body(p_ref, out_ref):
  m = jnp.max(p_ref[...], axis=0)
  out_ref[...] = jnp.where(m <= NEG_SENTINEL, 0.0, m)


@functools.lru_cache(maxsize=None)
def _get_segreduce():
  return pl.pallas_call(
      _segreduce_body,
      out_shape=jax.ShapeDtypeStruct((B, D), jnp.float32),
      name="segment_max_reduce")


def kernel(ent_embed, rel_embed, edge_index, edge_type, q_diameters,
           batch_idx, target_idx, W, W_loop, W_rel, loop_rel, bias,
           bn_gamma, bn_beta):
  src = edge_index[0].astype(jnp.int32).reshape(NW, NCH, C)
  dst = edge_index[1].astype(jnp.int32).reshape(NW, NCH, C)
  typ = edge_type.astype(jnp.int32).reshape(NW, NCH, C)
  idx_pack = jnp.stack([src, dst, typ], axis=2)
  z = jnp.zeros((N, D), jnp.float32)
  q2 = q_diameters.astype(jnp.int32).reshape(N, 1)
  lr = loop_rel.reshape(1, D)
  b2 = bias.reshape(1, D)
  g2 = bn_gamma.reshape(1, D)
  be2 = bn_beta.reshape(1, D)

  ent = ent_embed
  rel = rel_embed
  (deg16,) = _get_deg_kernel()(dst, z)
  for layer in range(3):
    (parts,) = _get_edge_kernel()(ent, rel, idx_pack, z)
    ent, rel = _dense_layers[layer](parts, deg16, ent, rel, q2,
                                    W, W_loop, W_rel, lr, b2, g2, be2)

  sent = jnp.full((64, D), NEG_SENTINEL, jnp.float32)
  (partials,) = _get_segmax_kernel()(ent, batch_idx.astype(jnp.int32), sent)
  return _get_segreduce()(partials)


# speedup vs baseline: 2.6163x; 1.0795x over previous
import functools

import jax
import jax.numpy as jnp
from jax import lax
from jax.experimental import pallas as pl
from jax.experimental.pallas import tpu as pltpu
from jax.experimental.pallas import tpu_sc as plsc

N = 10000
E = 320000
D = 128
R = 500
B = 1024

NC = 2
NS = 16
NW = NC * NS
EPT = E // NW
C = 50
NCH = EPT // C
ROWS_PT = 624
ROWS_LAST = N - ROWS_PT * (NS - 1)

NEG_SENTINEL = -3.0e38


def _edge_kernel_body(ent_hbm, rel_hbm, idx_hbm, z_hbm, out_hbm,
                      *refs):
  idxs = refs[0:6]
  rows = refs[6:9]
  relr = refs[9:11]
  agg_sp, rel_sp = refs[11], refs[12]
  semg = refs[13:16]
  semr = refs[16:18]
  sems = refs[18:21]
  semi = refs[21:27]

  cid = lax.axis_index("c")
  sid = lax.axis_index("s")
  w = sid * NC + cid

  @pl.when(sid == 0)
  def _():
    pltpu.sync_copy(rel_hbm, rel_sp)

  for q in range(4):
    pltpu.async_copy(idx_hbm.at[w, q], idxs[q], semi[q])

  def _per_tile_rows(fn):
    @pl.when(sid < NS - 1)
    def _():
      fn(sid * ROWS_PT, ROWS_PT)

    @pl.when(sid == NS - 1)
    def _():
      fn(ROWS_PT * (NS - 1), ROWS_LAST)

  def _zero(base, size):
    pltpu.sync_copy(z_hbm.at[pl.ds(base, size)],
                    agg_sp.at[pl.ds(base, size)])
  _per_tile_rows(_zero)

  for b in range(2):
    pltpu.make_async_copy(idx_hbm.at[w, b], idxs[b], semi[b]).wait()
    pltpu.async_copy(ent_hbm.at[idxs[b].at[0]], rows[b], semg[b])

  plsc.subcore_barrier()

  for b in range(2):
    pltpu.async_copy(rel_sp.at[idxs[b].at[2]], relr[b], semr[b])

  def compute_chunk(rows_ref, relr_ref):
    @plsc.parallel_loop(0, C, 1, unroll=4)
    def _(e):
      for k in range(8):
        sl = pl.ds(k * 16, 16)
        rows_ref[e, sl] = rows_ref[e, sl] * relr_ref[e, sl]

  def do_chunk(jb, g3, b2, q6):
    g3n = (g3 + 2) % 3
    g3p = (g3 + 2) % 3 if False else (g3 - 1) % 3
    q6n = (q6 + 2) % 6
    q6i = (q6 + 4) % 6
    pltpu.make_async_copy(ent_hbm.at[idxs[q6].at[0]], rows[g3],
                          semg[g3]).wait()
    pltpu.make_async_copy(rel_sp.at[idxs[q6].at[2]], relr[b2],
                          semr[b2]).wait()
    compute_chunk(rows[g3], relr[b2])
    pltpu.async_copy(rows[g3], agg_sp.at[idxs[q6].at[1]], sems[g3], add=True)

    @pl.when(jb >= 1)
    def _():
      jp = jb - 1
      pltpu.make_async_copy(rows[(g3 - 1) % 3],
                            agg_sp.at[idxs[(q6 - 1) % 6].at[1]],
                            sems[(g3 - 1) % 3]).wait()

    @pl.when(jb + 2 < NCH)
    def _():
      pltpu.make_async_copy(idx_hbm.at[w, jb + 2], idxs[q6n], semi[q6n]).wait()
      pltpu.async_copy(ent_hbm.at[idxs[q6n].at[0]], rows[g3n], semg[g3n])
      pltpu.async_copy(rel_sp.at[idxs[q6n].at[2]], relr[b2], semr[b2])

    @pl.when(jb + 4 < NCH)
    def _():
      pltpu.async_copy(idx_hbm.at[w, jb + 4], idxs[q6i], semi[q6i])

  def hex_body(p, _):
    jb = 6 * p
    for t in range(6):
      do_chunk(jb + t, t % 3, t % 2, t)
    return 0

  lax.fori_loop(0, NCH // 6, hex_body, 0)
  for t in range(NCH - (NCH // 6) * 6):
    jb = (NCH // 6) * 6 + t
    do_chunk(jb, jb % 3, jb % 2, jb % 6)

  jl = NCH - 1
  pltpu.make_async_copy(rows[jl % 3], agg_sp.at[idxs[jl % 6].at[1]],
                        sems[jl % 3]).wait()

  plsc.subcore_barrier()

  def _writeout(base, size):
    pltpu.sync_copy(agg_sp.at[pl.ds(base, size)],
                    out_hbm.at[cid, pl.ds(base, size)])
  _per_tile_rows(_writeout)


def _make_edge_kernel():
  mesh = plsc.VectorSubcoreMesh(core_axis_name="c", subcore_axis_name="s")
  scratch = (
      [pltpu.VMEM((3, C), jnp.int32) for _ in range(6)]
      + [pltpu.VMEM((C, D), jnp.float32) for _ in range(3)]
      + [pltpu.VMEM((C, D), jnp.float32) for _ in range(2)]
      + [pltpu.VMEM_SHARED((N, D), jnp.float32),
         pltpu.VMEM_SHARED((R, D), jnp.float32)]
      + [pltpu.SemaphoreType.DMA] * 14)
  return pl.kernel(
      _edge_kernel_body,
      out_type=[jax.ShapeDtypeStruct((NC, N, D), jnp.float32)],
      mesh=mesh, scratch_types=scratch, name="compgcn_edges")


def _deg_kernel_body(dst_hbm, z_hbm, deg_hbm, dst_v, ones_v, deg_sp,
                     *sems_s):
  cid = lax.axis_index("c")
  sid = lax.axis_index("s")
  w = sid * NC + cid

  pltpu.sync_copy(dst_hbm.at[w], dst_v)

  def fill_ones(i, _):
    for k in range(8):
      ones_v[i, pl.ds(k * 16, 16)] = jnp.full((16,), 1.0, jnp.float32)
    return 0
  lax.fori_loop(0, C, fill_ones, 0)

  def _per_tile_rows(fn):
    @pl.when(sid < NS - 1)
    def _():
      fn(sid * ROWS_PT, ROWS_PT)

    @pl.when(sid == NS - 1)
    def _():
      fn(ROWS_PT * (NS - 1), ROWS_LAST)

  def _zero(base, size):
    pltpu.sync_copy(z_hbm.at[pl.ds(base, size)],
                    deg_sp.at[pl.ds(base, size)])
  _per_tile_rows(_zero)
  plsc.subcore_barrier()

  NSEM = len(sems_s)

  def do_scatter(jb, q):
    @pl.when(jb >= NSEM)
    def _():
      pltpu.make_async_copy(ones_v, deg_sp.at[dst_v.at[jb - NSEM]],
                            sems_s[q]).wait()
    pltpu.async_copy(ones_v, deg_sp.at[dst_v.at[jb]], sems_s[q], add=True)

  def oct_body(p, _):
    for t in range(NSEM):
      do_scatter(NSEM * p + t, t)
    return 0

  lax.fori_loop(0, NCH // NSEM, oct_body, 0)
  for t in range(NCH - (NCH // NSEM) * NSEM):
    do_scatter((NCH // NSEM) * NSEM + t, t)
  for jb in range(NCH - NSEM, NCH):
    pltpu.make_async_copy(ones_v, deg_sp.at[dst_v.at[jb]],
                          sems_s[jb % NSEM]).wait()

  plsc.subcore_barrier()

  def _writeout(base, size):
    pltpu.sync_copy(deg_sp.at[pl.ds(base, size)],
                    deg_hbm.at[cid, pl.ds(base, size)])
  _per_tile_rows(_writeout)


@functools.lru_cache(maxsize=None)
def _get_deg_kernel():
  mesh = plsc.VectorSubcoreMesh(core_axis_name="c", subcore_axis_name="s")
  return pl.kernel(
      _deg_kernel_body,
      out_type=[jax.ShapeDtypeStruct((NC, N, D), jnp.float32)],
      mesh=mesh,
      scratch_types=[
          pltpu.VMEM((NCH, C), jnp.int32),
          pltpu.VMEM((C, D), jnp.float32),
          pltpu.VMEM_SHARED((N, D), jnp.float32),
      ] + [pltpu.SemaphoreType.DMA] * 8,
      name="compgcn_deg")


@functools.lru_cache(maxsize=None)
def _get_edge_kernel():
  return _make_edge_kernel()


def _dense_body(layer, do_relu, p_ref, deg_ref, ent_ref, rel_ref, q_ref,
                w_ref, wl_ref, wr_ref, lr_ref, b_ref, g_ref, be_ref,
                oent_ref, orel_ref):
  pre = p_ref[0] + p_ref[1]
  deg = deg_ref[0, :, 0:1] + deg_ref[1, :, 0:1]
  agg = jnp.dot(pre, w_ref[...], preferred_element_type=jnp.float32)
  agg = agg / jnp.maximum(deg, 1.0)
  loop_msg = jnp.dot(ent_ref[...] * lr_ref[...], wl_ref[...],
                     preferred_element_type=jnp.float32)
  out = 0.5 * (agg + loop_msg) + b_ref[...]
  m = jnp.mean(out, axis=0, keepdims=True)
  v = jnp.mean((out - m) * (out - m), axis=0, keepdims=True)
  out = g_ref[...] * (out - m) * lax.rsqrt(v + 1e-5) + be_ref[...]
  out = jnp.where(q_ref[...] <= layer, ent_ref[...], out)
  if do_relu:
    out = jnp.maximum(out, 0.0)
  oent_ref[...] = out
  orel_ref[...] = jnp.dot(rel_ref[...], wr_ref[...],
                          preferred_element_type=jnp.float32)


def _make_dense(layer, do_relu):
  return pl.pallas_call(
      functools.partial(_dense_body, layer, do_relu),
      out_shape=[jax.ShapeDtypeStruct((N, D), jnp.float32),
                 jax.ShapeDtypeStruct((R, D), jnp.float32)],
      name=f"compgcn_dense_{layer}")


_dense_layers = [_make_dense(0, True), _make_dense(1, True),
                 _make_dense(2, False)]

SEG_ROWS = 320
SEG_GROUPS = SEG_ROWS // 16


def _segmax_body(ent_hbm, bid_hbm, sent_hbm, out_hbm,
                 bid_v, rows_v, stag_v, sent_v):
  cid = lax.axis_index("c")
  sid = lax.axis_index("s")
  w = sid * NC + cid
  base = w * SEG_ROWS
  is_last = w == NW - 1
  mysize = jnp.where(is_last, N - (NW - 1) * SEG_ROWS, SEG_ROWS)
  ngroups = mysize // 16

  pltpu.sync_copy(sent_hbm, sent_v)
  for blk in range(B // 64):
    pltpu.sync_copy(sent_v, out_hbm.at[w, pl.ds(blk * 64, 64)])

  @pl.when(jnp.logical_not(is_last))
  def _():
    pltpu.sync_copy(bid_hbm.at[pl.ds(base, SEG_ROWS)], bid_v)
    pltpu.sync_copy(ent_hbm.at[pl.ds(base, SEG_ROWS)], rows_v)

  @pl.when(is_last)
  def _():
    tail = N - (NW - 1) * SEG_ROWS
    pltpu.sync_copy(bid_hbm.at[pl.ds(base, tail)], bid_v.at[pl.ds(0, tail)])
    pltpu.sync_copy(ent_hbm.at[pl.ds(base, tail)], rows_v.at[pl.ds(0, tail)])

  for k in range(8):
    stag_v[pl.ds(k * 16, 16)] = jnp.full((16,), NEG_SENTINEL, jnp.float32)

  prev0 = bid_v[pl.ds(0, 16)][0]

  def group_scan(g, prev):
    ids16 = bid_v[pl.ds(g * 16, 16)]

    def do_lane(lane, prev_lane):
      rid = ids16[lane]
      changed = jnp.logical_and(rid != prev_lane, g * 16 + lane < mysize)

      @pl.when(changed)
      def _():
        pltpu.sync_copy(stag_v, out_hbm.at[w, prev_lane])

      @pl.when(g * 16 + lane < mysize)
      def _():
        r = g * 16 + lane
        for k in range(8):
          sl = pl.ds(k * 16, 16)
          row = rows_v[r, sl]
          cur = stag_v[sl]
          stag_v[sl] = jnp.where(changed, row, jnp.maximum(cur, row))
      return jnp.where(g * 16 + lane < mysize, rid, prev_lane)

    for lane in range(16):
      prev = do_lane(lane, prev)
    return prev

  prev_last = lax.fori_loop(0, SEG_GROUPS, group_scan, prev0)
  pltpu.sync_copy(stag_v, out_hbm.at[w, prev_last])


@functools.lru_cache(maxsize=None)
def _get_segmax_kernel():
  return pl.kernel(
      _segmax_body,
      out_type=[jax.ShapeDtypeStruct((NW, B, D), jnp.float32)],
      mesh=plsc.VectorSubcoreMesh(core_axis_name="c", subcore_axis_name="s"),
      scratch_types=[
          pltpu.VMEM((SEG_ROWS,), jnp.int32),
          pltpu.VMEM((SEG_ROWS, D), jnp.float32),
          pltpu.VMEM((D,), jnp.float32),
          pltpu.VMEM((64, D), jnp.float32),
      ],
      name="segment_max_partials")


def _segreduce_body(p_ref, out_ref):
  m = jnp.max(p_ref[...], axis=0)
  out_ref[...] = jnp.where(m <= NEG_SENTINEL, 0.0, m)


@functools.lru_cache(maxsize=None)
def _get_segreduce():
  return pl.pallas_call(
      _segreduce_body,
      out_shape=jax.ShapeDtypeStruct((B, D), jnp.float32),
      name="segment_max_reduce")


def kernel(ent_embed, rel_embed, edge_index, edge_type, q_diameters,
           batch_idx, target_idx, W, W_loop, W_rel, loop_rel, bias,
           bn_gamma, bn_beta):
  src = edge_index[0].astype(jnp.int32).reshape(NW, NCH, C)
  dst = edge_index[1].astype(jnp.int32).reshape(NW, NCH, C)
  typ = edge_type.astype(jnp.int32).reshape(NW, NCH, C)
  idx_pack = jnp.stack([src, dst, typ], axis=2)
  z = jnp.zeros((N, D), jnp.float32)
  q2 = q_diameters.astype(jnp.int32).reshape(N, 1)
  lr = loop_rel.reshape(1, D)
  b2 = bias.reshape(1, D)
  g2 = bn_gamma.reshape(1, D)
  be2 = bn_beta.reshape(1, D)

  ent = ent_embed
  rel = rel_embed
  (deg16,) = _get_deg_kernel()(dst, z)
  for layer in range(3):
    (parts,) = _get_edge_kernel()(ent, rel, idx_pack, z)
    ent, rel = _dense_layers[layer](parts, deg16, ent, rel, q2,
                                    W, W_loop, W_rel, lr, b2, g2, be2)

  sent = jnp.full((64, D), NEG_SENTINEL, jnp.float32)
  (partials,) = _get_segmax_kernel()(ent, batch_idx.astype(jnp.int32), sent)
  return _get_segreduce()(partials)
